# baseline (speedup 1.0000x reference)
import functools

import numpy as np
import jax
import jax.numpy as jnp
from jax import lax
from jax.experimental import pallas as pl
from jax.experimental.pallas import tpu as pltpu
from jax.experimental.pallas import tpu_sc as plsc

C = 200
CP = 208
IH, IW = 113, 200
OH, OW = 225, 400
HW = OH * OW
ZERO_ROW = HW
OHP = 240
TROWS = OHP * OW
N = 262144

NC, NS = 2, 16
NW = NC * NS
BW = N // NW
CH = 128
NCH = BW // CH
SUP = 2048
F32 = jnp.float32


def _wmat(in_size, out_size, out_pad):
    scale = out_size / in_size
    sample_f = (np.arange(out_size, dtype=np.float64) + 0.5) / scale - 0.5
    x = np.abs(sample_f[None, :] - np.arange(in_size, dtype=np.float64)[:, None])
    w = np.maximum(0.0, 1.0 - x)
    total = w.sum(axis=0, keepdims=True)
    w = np.where(total > 0, w / total, 0)
    out = np.zeros((out_pad, in_size), dtype=np.float32)
    out[:out_size, :] = w.T.astype(np.float32)
    return out

IHP = 120
_BI = 8

_AH = _wmat(IH, OH, OHP)
_AHP = np.zeros((OHP, IHP), dtype=np.float32)
_AHP[:, :IH] = _AH
_AW = _wmat(IW, OW, OW)

_HI = jax.lax.Precision.HIGHEST


def _a1_body(aw_ref, x_ref, o_ref):
    i = pl.program_id(0)
    for di in range(_BI):
        blk = x_ref[:, di, :]
        res = lax.dot_general(
            aw_ref[...], blk, (((1,), (1,)), ((), ())),
            preferred_element_type=F32, precision=_HI)
        o_ref[di, :, :C] = res
        o_ref[di, :, C:] = jnp.zeros((OW, CP - C), F32)
    @pl.when(i == IHP // _BI - 1)
    def _():
        o_ref[pl.ds(1, _BI - 1)] = jnp.zeros((_BI - 1, OW, CP), F32)


_a1 = pl.pallas_call(
    _a1_body,
    grid=(IHP // _BI,),
    in_specs=[
        pl.BlockSpec((OW, IW), lambda i: (0, 0)),
        pl.BlockSpec((C, _BI, IW), lambda i: (0, i, 0)),
    ],
    out_specs=pl.BlockSpec((_BI, OW, CP), lambda i: (i, 0, 0)),
    out_shape=jax.ShapeDtypeStruct((IHP, OW, CP), F32),
)


_KB = 3328


def _a2_body(ah_ref, u_ref, o_ref):
    o_ref[...] = lax.dot_general(
        ah_ref[...], u_ref[...], (((1,), (0,)), ((), ())),
        preferred_element_type=F32, precision=_HI)


_a2 = pl.pallas_call(
    _a2_body,
    grid=(OW * CP // _KB,),
    in_specs=[
        pl.BlockSpec((OHP, IHP), lambda k: (0, 0)),
        pl.BlockSpec((IHP, _KB), lambda k: (0, k)),
    ],
    out_specs=pl.BlockSpec((OHP, _KB), lambda k: (0, k)),
    out_shape=jax.ShapeDtypeStruct((OHP, OW * CP), F32),
)


_NB = 4


def _sc_body(table, pixx, pixy, maski, outa, outb,
             pixx_v, pixy_v, mask_v, idx_v,
             rows0, rows1, rows2, rows3,
             g0, g1, g2, g3, w0, w1, w2, w3, v0, v1, v2, v3):
    wid = lax.axis_index("s") * NC + lax.axis_index("c")
    base = wid * BW

    with jax.named_scope("idx_phase"):
      for sc in range(BW // SUP):
        pltpu.sync_copy(pixx.at[pl.ds(base + sc * SUP, SUP)], pixx_v)
        pltpu.sync_copy(pixy.at[pl.ds(base + sc * SUP, SUP)], pixy_v)
        pltpu.sync_copy(maski.at[pl.ds(base + sc * SUP, SUP)], mask_v)

        def _idx_step(i, _, sc=sc):
            x = pixx_v[pl.ds(i * 16, 16)]
            y = pixy_v[pl.ds(i * 16, 16)]
            m = mask_v[pl.ds(i * 16, 16)]
            pad = ZERO_ROW + wid * 128 + (
                (i * 16 + lax.iota(jnp.int32, 16)) & 127)
            idx = jnp.where(m != 0, y * OW + x, pad)
            idx_v[pl.ds(sc * SUP + i * 16, 16)] = idx
            return 0

        lax.fori_loop(0, SUP // 16, _idx_step, 0)

    rows_sl = (rows0, rows1, rows2, rows3)
    gsem = (g0, g1, g2, g3)
    wsem = (w0, w1, w2, w3)
    vsem = (v0, v1, v2, v3)

    def _gissue(k, s):
        pltpu.async_copy(table.at[idx_v.at[pl.ds(k * CH, CH)]],
                         rows_sl[s], gsem[s])

    def _gwait(s):
        pltpu.make_async_copy(table.at[pl.ds(0, CH)],
                              rows_sl[s], gsem[s]).wait()

    def _wissue(k, s):
        pltpu.async_copy(rows_sl[s].at[:, pl.ds(0, 128)],
                         outa.at[pl.ds(base + k * CH, CH)], wsem[s])
        pltpu.async_copy(rows_sl[s].at[:, pl.ds(80, 128)],
                         outb.at[pl.ds(base + k * CH, CH)], vsem[s])

    def _wwait(s):
        pltpu.make_async_copy(rows_sl[s].at[:, pl.ds(0, 128)],
                              outa.at[pl.ds(0, CH)], wsem[s]).wait()
        pltpu.make_async_copy(rows_sl[s].at[:, pl.ds(80, 128)],
                              outb.at[pl.ds(0, CH)], vsem[s]).wait()

    with jax.named_scope("gather_phase"):
      for k in range(_NB):
        _gissue(k, k)
        if k >= 2:
            _gwait(k - 2)
            _wissue(k - 2, k - 2)

      def _quad(t, _):
        for s in range(_NB):
            k = 4 * t + s
            s2 = (s + 2) % _NB
            _wwait(s)
            _gissue(k, s)
            _gwait(s2)
            _wissue(k - 2, s2)
        return 0

      lax.fori_loop(1, NCH // _NB, _quad, 0)
      for j in range(NCH - 2, NCH):
        sj = j % _NB
        _gwait(sj)
        _wissue(j, sj)
      for s in range(_NB):
        _wwait(s)


@functools.lru_cache(maxsize=1)
def _get_sc_gather():
    return pl.kernel(
        _sc_body,
        out_type=[jax.ShapeDtypeStruct((N, 128), F32),
                  jax.ShapeDtypeStruct((N, 128), F32)],
        mesh=plsc.VectorSubcoreMesh(core_axis_name="c", subcore_axis_name="s",
                                    num_cores=NC, num_subcores=NS),
        compiler_params=pltpu.CompilerParams(use_tc_tiling_on_sc=False),
        scratch_types=[
            pltpu.VMEM((SUP,), jnp.int32),
            pltpu.VMEM((SUP,), jnp.int32),
            pltpu.VMEM((SUP,), jnp.int32),
            pltpu.VMEM((BW,), jnp.int32),
            pltpu.VMEM((CH, CP), F32),
            pltpu.VMEM((CH, CP), F32),
            pltpu.VMEM((CH, CP), F32),
            pltpu.VMEM((CH, CP), F32),
        ] + [pltpu.SemaphoreType.DMA] * 12,
    )


_BN = 512


def _c_body(ga_ref, gb_ref, o_ref):
    ta = jnp.transpose(ga_ref[...], (1, 0))
    tb = jnp.transpose(gb_ref[...], (1, 0))
    o_ref[0:128, 0, :, :] = ta.reshape(128, _BN // 16, 16)
    o_ref[128:200, 0, :, :] = tb[48:120, :].reshape(72, _BN // 16, 16)


_c_t = pl.pallas_call(
    _c_body,
    grid=(N // _BN,),
    in_specs=[pl.BlockSpec((_BN, 128), lambda b: (b, 0)),
              pl.BlockSpec((_BN, 128), lambda b: (b, 0))],
    out_specs=pl.BlockSpec((C, 1, _BN // 16, 16),
                           lambda b: (0, b // (2048 // _BN),
                                      b % (2048 // _BN), 0)),
    out_shape=jax.ShapeDtypeStruct((C, 128, 128, 16), F32),
)


def kernel(x2d, pix, mask):
    u = _a1(jnp.asarray(_AW), x2d)
    t = _a2(jnp.asarray(_AHP), u.reshape(IHP, OW * CP))
    table = t.reshape(TROWS, CP)
    ga, gb = _get_sc_gather()(table, pix[:, 0], pix[:, 1],
                              mask.astype(jnp.int32))
    return _c_t(ga, gb)


# speedup vs baseline: 1.7683x; 1.7683x over previous
import functools

import numpy as np
import jax
import jax.numpy as jnp
from jax import lax
from jax.experimental import pallas as pl
from jax.experimental.pallas import tpu as pltpu
from jax.experimental.pallas import tpu_sc as plsc

C = 200
CP = 208
IH, IW = 113, 200
OH, OW = 225, 400
HW = OH * OW
ZERO_ROW = HW
OHP = 240
TROWS = OHP * OW
N = 262144

NC, NS = 2, 16
NW = NC * NS
BW = N // NW
CH = 128
NCH = BW // CH
SUP = 2048
F32 = jnp.float32


def _wmat(in_size, out_size, out_pad):
    scale = out_size / in_size
    sample_f = (np.arange(out_size, dtype=np.float64) + 0.5) / scale - 0.5
    x = np.abs(sample_f[None, :] - np.arange(in_size, dtype=np.float64)[:, None])
    w = np.maximum(0.0, 1.0 - x)
    total = w.sum(axis=0, keepdims=True)
    w = np.where(total > 0, w / total, 0)
    out = np.zeros((out_pad, in_size), dtype=np.float32)
    out[:out_size, :] = w.T.astype(np.float32)
    return out

IHP = 120
_BI = 8

_AH = _wmat(IH, OH, OHP)
_AHP = np.zeros((OHP, IHP), dtype=np.float32)
_AHP[:, :IH] = _AH
_AW = _wmat(IW, OW, OW)

_HI = jax.lax.Precision.HIGHEST


def _a1_body(aw_ref, x_ref, o_ref):
    i = pl.program_id(0)
    for di in range(_BI):
        blk = x_ref[:, di, :]
        res = lax.dot_general(
            aw_ref[...], blk, (((1,), (1,)), ((), ())),
            preferred_element_type=F32, precision=_HI)
        o_ref[di, :, :C] = res
        o_ref[di, :, C:] = jnp.zeros((OW, CP - C), F32)
    @pl.when(i == IHP // _BI - 1)
    def _():
        o_ref[pl.ds(1, _BI - 1)] = jnp.zeros((_BI - 1, OW, CP), F32)


_a1 = pl.pallas_call(
    _a1_body,
    grid=(IHP // _BI,),
    in_specs=[
        pl.BlockSpec((OW, IW), lambda i: (0, 0)),
        pl.BlockSpec((C, _BI, IW), lambda i: (0, i, 0)),
    ],
    out_specs=pl.BlockSpec((_BI, OW, CP), lambda i: (i, 0, 0)),
    out_shape=jax.ShapeDtypeStruct((IHP, OW, CP), F32),
)


_KB = 3328


def _a2_body(ah_ref, u_ref, o_ref):
    o_ref[...] = lax.dot_general(
        ah_ref[...], u_ref[...], (((1,), (0,)), ((), ())),
        preferred_element_type=F32, precision=_HI)


_a2 = pl.pallas_call(
    _a2_body,
    grid=(OW * CP // _KB,),
    in_specs=[
        pl.BlockSpec((OHP, IHP), lambda k: (0, 0)),
        pl.BlockSpec((IHP, _KB), lambda k: (0, k)),
    ],
    out_specs=pl.BlockSpec((OHP, _KB), lambda k: (0, k)),
    out_shape=jax.ShapeDtypeStruct((OHP, OW * CP), F32),
)


_NB = 4


def _sc_body(table, pixx, pixy, maski, outa, outb,
             pixx_v, pixy_v, mask_v, idx_v,
             rows0, rows1, rows2, rows3,
             g0, g1, g2, g3, w0, w1, w2, w3, v0, v1, v2, v3):
    wid = lax.axis_index("s") * NC + lax.axis_index("c")
    base = wid * BW

    with jax.named_scope("idx_phase"):
      for sc in range(BW // SUP):
        pltpu.sync_copy(pixx.at[pl.ds(base + sc * SUP, SUP)], pixx_v)
        pltpu.sync_copy(pixy.at[pl.ds(base + sc * SUP, SUP)], pixy_v)
        pltpu.sync_copy(maski.at[pl.ds(base + sc * SUP, SUP)], mask_v)

        def _idx_step(i, _, sc=sc):
            x = pixx_v[pl.ds(i * 16, 16)]
            y = pixy_v[pl.ds(i * 16, 16)]
            m = mask_v[pl.ds(i * 16, 16)]
            pad = ZERO_ROW + wid * 128 + (
                (i * 16 + lax.iota(jnp.int32, 16)) & 127)
            idx = jnp.where(m != 0, y * OW + x, pad)
            idx_v[pl.ds(sc * SUP + i * 16, 16)] = idx
            return 0

        lax.fori_loop(0, SUP // 16, _idx_step, 0)

    rows_sl = (rows0, rows1, rows2, rows3)
    gsem = (g0, g1, g2, g3)
    wsem = (w0, w1, w2, w3)
    vsem = (v0, v1, v2, v3)

    def _gissue(k, s):
        pltpu.async_copy(table.at[idx_v.at[pl.ds(k * CH, CH)]],
                         rows_sl[s], gsem[s])

    def _gwait(s):
        pltpu.make_async_copy(table.at[pl.ds(0, CH)],
                              rows_sl[s], gsem[s]).wait()

    def _wissue(k, s):
        pltpu.async_copy(rows_sl[s].at[:, pl.ds(0, 128)],
                         outa.at[pl.ds(base + k * CH, CH)], wsem[s])
        pltpu.async_copy(rows_sl[s].at[:, pl.ds(80, 128)],
                         outb.at[pl.ds(base + k * CH, CH)], vsem[s])

    def _wwait(s):
        pltpu.make_async_copy(rows_sl[s].at[:, pl.ds(0, 128)],
                              outa.at[pl.ds(0, CH)], wsem[s]).wait()
        pltpu.make_async_copy(rows_sl[s].at[:, pl.ds(80, 128)],
                              outb.at[pl.ds(0, CH)], vsem[s]).wait()

    with jax.named_scope("gather_phase"):
      for k in range(_NB):
        _gissue(k, k)
        if k >= 2:
            _gwait(k - 2)
            _wissue(k - 2, k - 2)

      def _quad(t, _):
        for s in range(_NB):
            k = 4 * t + s
            s2 = (s + 2) % _NB
            _wwait(s)
            _gissue(k, s)
            _gwait(s2)
            _wissue(k - 2, s2)
        return 0

      lax.fori_loop(1, NCH // _NB, _quad, 0)
      for j in range(NCH - 2, NCH):
        sj = j % _NB
        _gwait(sj)
        _wissue(j, sj)
      for s in range(_NB):
        _wwait(s)


@functools.lru_cache(maxsize=1)
def _get_sc_gather():
    return pl.kernel(
        _sc_body,
        out_type=[jax.ShapeDtypeStruct((N, 128), F32),
                  jax.ShapeDtypeStruct((N, 128), F32)],
        mesh=plsc.VectorSubcoreMesh(core_axis_name="c", subcore_axis_name="s",
                                    num_cores=NC, num_subcores=NS),
        compiler_params=pltpu.CompilerParams(use_tc_tiling_on_sc=False),
        scratch_types=[
            pltpu.VMEM((SUP,), jnp.int32),
            pltpu.VMEM((SUP,), jnp.int32),
            pltpu.VMEM((SUP,), jnp.int32),
            pltpu.VMEM((BW,), jnp.int32),
            pltpu.VMEM((CH, CP), F32),
            pltpu.VMEM((CH, CP), F32),
            pltpu.VMEM((CH, CP), F32),
            pltpu.VMEM((CH, CP), F32),
        ] + [pltpu.SemaphoreType.DMA] * 12,
    )


_BN = 512


def _c_body(ga_ref, gb_ref, o_ref):
    o_ref[0:128, :] = jnp.transpose(ga_ref[...], (1, 0))
    tb = jnp.transpose(gb_ref[...], (1, 0))
    o_ref[128:200, :] = tb[48:120, :]


_c_t = pl.pallas_call(
    _c_body,
    grid=(N // _BN,),
    in_specs=[pl.BlockSpec((_BN, 128), lambda b: (b, 0)),
              pl.BlockSpec((_BN, 128), lambda b: (b, 0))],
    out_specs=pl.BlockSpec((C, _BN), lambda b: (0, b)),
    out_shape=jax.ShapeDtypeStruct((C, N), F32),
)


def kernel(x2d, pix, mask):
    u = _a1(jnp.asarray(_AW), x2d)
    t = _a2(jnp.asarray(_AHP), u.reshape(IHP, OW * CP))
    table = t.reshape(TROWS, CP)
    ga, gb = _get_sc_gather()(table, pix[:, 0], pix[:, 1],
                              mask.astype(jnp.int32))
    out = _c_t(ga, gb)
    return out.reshape(C, 128, 128, 16)
